# baseline (speedup 1.0000x reference)
import functools
import math

import jax
import jax.numpy as jnp
from jax import lax
from jax.experimental import pallas as pl
from jax.experimental.pallas import tpu as pltpu
from jax.experimental.pallas import tpu_sc as plsc

_NCLASSES = 20
_SAMPLES = 128
_BP = 2048
_NW = 32
_GPW = (_NCLASSES * _SAMPLES) // _NW
_PSTRIDE = 131072
_CH = 128


def _gather_body(topi_hbm, fs_hbm, ft_hbm, out_s, out_t,
                 idx_v, bases_v, chidx_v, rows_s, rows_t, sem_s, sem_t):
    wid = lax.axis_index("s") * 2 + lax.axis_index("c")
    base = wid * _GPW
    pltpu.sync_copy(topi_hbm.at[pl.ds(base, _GPW)], idx_v)
    lane = lax.iota(jnp.int32, 16)
    for t in range(_GPW // 16):
        nvec = idx_v[pl.ds(t * 16, 16)]
        bvec = nvec + lax.shift_right_logical(nvec, 17) * ((_CH - 1) * _PSTRIDE)
        for b in range(16):
            j = t * 16 + b
            bj = jnp.broadcast_to(
                jnp.sum(jnp.where(lane == b, bvec, 0)), (16,))
            for g in range(8):
                chidx_v[j, pl.ds(g * 16, 16)] = (
                    bj + (lane + g * 16) * _PSTRIDE)
    for j in range(_GPW):
        pltpu.async_copy(fs_hbm.at[chidx_v.at[j]], rows_s.at[j], sem_s)
        pltpu.async_copy(ft_hbm.at[chidx_v.at[j]], rows_t.at[j], sem_t)
    for j in range(_GPW):
        pltpu.make_async_copy(fs_hbm.at[chidx_v.at[j]], rows_s.at[j],
                              sem_s).wait()
        pltpu.make_async_copy(ft_hbm.at[chidx_v.at[j]], rows_t.at[j],
                              sem_t).wait()
    pltpu.sync_copy(rows_s, out_s.at[pl.ds(base, _GPW)])
    pltpu.sync_copy(rows_t, out_t.at[pl.ds(base, _GPW)])


def _sc_gather(topi_flat, fs_flat, ft_flat):
    n_samp = _NCLASSES * _SAMPLES
    mesh = plsc.VectorSubcoreMesh(core_axis_name="c", subcore_axis_name="s")
    f = pl.kernel(
        _gather_body,
        mesh=mesh,
        compiler_params=pltpu.CompilerParams(needs_layout_passes=False),
        out_type=[
            jax.ShapeDtypeStruct((n_samp, _CH), jnp.float32),
            jax.ShapeDtypeStruct((n_samp, _CH), jnp.float32),
        ],
        scratch_types=[
            pltpu.VMEM((_GPW,), jnp.int32),
            pltpu.VMEM((_GPW,), jnp.int32),
            pltpu.VMEM((_GPW, _CH), jnp.int32),
            pltpu.VMEM((_GPW, _CH), jnp.float32),
            pltpu.VMEM((_GPW, _CH), jnp.float32),
            pltpu.SemaphoreType.DMA,
            pltpu.SemaphoreType.DMA,
        ],
    )
    return f(topi_flat, fs_flat, ft_flat)


_NBKT = 1024
_HISTW = _NCLASSES * _NBKT
_SLICE = 262144 // _NW
_CHUNK = 2048
_STRIPE = _HISTW // 16
_SH1 = 32 - 10
_SH2 = 32 - 20
_CCAP = 16
_LCAP = 240


def _sortable_key(s):
    k1 = plsc.bitcast(s, jnp.int32)
    return jnp.where(k1 < 0, ~k1, k1 | jnp.int32(-(2 ** 31)))


def _load_chunk(scores_hbm, labels_hbm, sc_v, lb_v, base):
    pltpu.sync_copy(scores_hbm.at[pl.ds(base, _CHUNK)], sc_v)
    pltpu.sync_copy(labels_hbm.at[pl.ds(base, _CHUNK)], lb_v)


def _zero_vmem_i32(ref, nwords):
    z = jnp.zeros((16,), jnp.int32)

    def b(i, c):
        ref[pl.ds(i * 16, 16)] = z
        return c

    lax.fori_loop(0, nwords // 16, b, 0)


def _publish_reduce(hist_v, spmem, red_v, tmp_v, sid, cid, out_hbm):
    pltpu.sync_copy(hist_v, spmem.at[sid])
    plsc.subcore_barrier()
    pltpu.sync_copy(spmem.at[0, pl.ds(sid * _STRIPE, _STRIPE)], red_v)
    for r in range(1, 16):
        pltpu.sync_copy(spmem.at[r, pl.ds(sid * _STRIPE, _STRIPE)], tmp_v)

        def badd(i, c):
            red_v[pl.ds(i * 16, 16)] = (red_v[pl.ds(i * 16, 16)]
                                        + tmp_v[pl.ds(i * 16, 16)])
            return c

        lax.fori_loop(0, _STRIPE // 16, badd, 0)
    pltpu.sync_copy(red_v, out_hbm.at[cid, pl.ds(sid * _STRIPE, _STRIPE)])


def _scan_boundary(hist_ref, k, target):
    lane = lax.iota(jnp.int32, 16)

    def cond(st):
        return jnp.logical_not(st[4])

    def body(st):
        v, cum, bb, aa, done = st
        hv = hist_ref[pl.ds(k * _NBKT + v * 16, 16)]
        rc = lax.rev(hv, (0,))
        cs = jnp.cumsum(rc)
        cum_incl = cum + cs
        mask = cum_incl >= target
        mask = jnp.logical_or(mask, jnp.logical_and(v == 0, lane == 15))
        anyhit = jnp.max(mask.astype(jnp.int32)) > 0
        f = jnp.max(plsc.all_reduce_ffs(mask))
        bnew = v * 16 + 15 - f
        csm1 = jnp.sum(jnp.where(lane == f - 1, cs, 0))
        anew = cum + csm1
        vec_total = jnp.sum(hv)
        return (jnp.where(anyhit, v, v - 1),
                jnp.where(anyhit, cum, cum + vec_total),
                jnp.where(anyhit, bnew, bb),
                jnp.where(anyhit, anew, aa),
                anyhit)

    st = lax.while_loop(cond, body, (jnp.int32(_NBKT // 16 - 1),
                                     jnp.int32(0), jnp.int32(0),
                                     jnp.int32(0), False))
    return st[2], st[3]


def _hist1_body(scores_hbm, labels_hbm, out_hbm,
                sc_v, lb_v, hist_v, red_v, tmp_v, spmem):
    cid = lax.axis_index("c")
    sid = lax.axis_index("s")
    wid = cid * 16 + sid
    _zero_vmem_i32(hist_v, _HISTW)
    ones = jnp.ones((16,), jnp.int32)
    for ch in range(_SLICE // _CHUNK):
        _load_chunk(scores_hbm, labels_hbm, sc_v, lb_v,
                    wid * _SLICE + ch * _CHUNK)

        def b(i, c):
            s = sc_v[pl.ds(i * 16, 16)]
            l = lb_v[pl.ds(i * 16, 16)]
            key = _sortable_key(s)
            b1 = lax.shift_right_logical(key, _SH1)
            plsc.addupdate_scatter(hist_v, [l * _NBKT + b1], ones)
            return c

        lax.fori_loop(0, _CHUNK // 16, b, 0)
    _publish_reduce(hist_v, spmem, red_v, tmp_v, sid, cid, out_hbm)


def _hist2_body(scores_hbm, labels_hbm, h1_hbm, out_hbm, ba_hbm,
                sc_v, lb_v, hist_v, red_v, tmp_v, h1s_v, btab_v, ba_v, spmem):
    cid = lax.axis_index("c")
    sid = lax.axis_index("s")
    wid = cid * 16 + sid
    lane = lax.iota(jnp.int32, 16)
    for part in range(16):
        pltpu.sync_copy(h1_hbm.at[0, pl.ds(part * _STRIPE, _STRIPE)], red_v)
        pltpu.sync_copy(h1_hbm.at[1, pl.ds(part * _STRIPE, _STRIPE)], tmp_v)

        def badd(i, c, _part=part):
            h1s_v[pl.ds(_part * _STRIPE + i * 16, 16)] = (
                red_v[pl.ds(i * 16, 16)] + tmp_v[pl.ds(i * 16, 16)])
            return c

        lax.fori_loop(0, _STRIPE // 16, badd, 0)
    b_lo = jnp.zeros((16,), jnp.int32)
    b_hi = jnp.zeros((16,), jnp.int32)
    a_lo = jnp.zeros((16,), jnp.int32)
    a_hi = jnp.zeros((16,), jnp.int32)
    for k in range(_NCLASSES):
        bk, ak = _scan_boundary(h1s_v, k, _SAMPLES)
        if k < 16:
            b_lo = jnp.where(lane == k, bk, b_lo)
            a_lo = jnp.where(lane == k, ak, a_lo)
        else:
            b_hi = jnp.where(lane == (k - 16), bk, b_hi)
            a_hi = jnp.where(lane == (k - 16), ak, a_hi)
    btab_v[pl.ds(0, 16)] = b_lo
    btab_v[pl.ds(16, 16)] = b_hi
    ba_v[pl.ds(0, 16)] = b_lo
    ba_v[pl.ds(16, 16)] = b_hi
    ba_v[pl.ds(32, 16)] = a_lo
    ba_v[pl.ds(48, 16)] = a_hi

    @pl.when(wid == 0)
    def _():
        pltpu.sync_copy(ba_v, ba_hbm)

    _zero_vmem_i32(hist_v, _HISTW)
    ones = jnp.ones((16,), jnp.int32)
    for ch in range(_SLICE // _CHUNK):
        _load_chunk(scores_hbm, labels_hbm, sc_v, lb_v,
                    wid * _SLICE + ch * _CHUNK)

        def b(i, c):
            s = sc_v[pl.ds(i * 16, 16)]
            l = lb_v[pl.ds(i * 16, 16)]
            key = _sortable_key(s)
            b1 = lax.shift_right_logical(key, _SH1)
            sub = jnp.bitwise_and(lax.shift_right_logical(key, _SH2),
                                  jnp.int32(_NBKT - 1))
            bl = plsc.load_gather(btab_v, [l])
            m = b1 == bl
            plsc.addupdate_scatter(hist_v, [l * _NBKT + sub], ones, mask=m)
            return c

        lax.fori_loop(0, _CHUNK // 16, b, 0)
    _publish_reduce(hist_v, spmem, red_v, tmp_v, sid, cid, out_hbm)


def _collect_body(scores_hbm, labels_hbm, h2_hbm, ba_hbm,
                  cs_hbm, ci_hbm,
                  sc_v, lb_v, red_v, tmp_v, h2s_v, ba_v, t22_v,
                  cmp_s, cmp_i, cmp_l, loc_s, loc_i, cnt_v):
    cid = lax.axis_index("c")
    sid = lax.axis_index("s")
    wid = cid * 16 + sid
    lane = lax.iota(jnp.int32, 16)
    for part in range(16):
        pltpu.sync_copy(h2_hbm.at[0, pl.ds(part * _STRIPE, _STRIPE)], red_v)
        pltpu.sync_copy(h2_hbm.at[1, pl.ds(part * _STRIPE, _STRIPE)], tmp_v)

        def badd(i, c, _part=part):
            h2s_v[pl.ds(_part * _STRIPE + i * 16, 16)] = (
                red_v[pl.ds(i * 16, 16)] + tmp_v[pl.ds(i * 16, 16)])
            return c

        lax.fori_loop(0, _STRIPE // 16, badd, 0)
    pltpu.sync_copy(ba_hbm, ba_v)
    b_lo = ba_v[pl.ds(0, 16)]
    b_hi = ba_v[pl.ds(16, 16)]
    a_lo = ba_v[pl.ds(32, 16)]
    a_hi = ba_v[pl.ds(48, 16)]
    t_lo = jnp.zeros((16,), jnp.int32)
    t_hi = jnp.zeros((16,), jnp.int32)
    for k in range(_NCLASSES):
        if k < 16:
            ak = jnp.sum(jnp.where(lane == k, a_lo, 0))
            bk = jnp.sum(jnp.where(lane == k, b_lo, 0))
        else:
            ak = jnp.sum(jnp.where(lane == (k - 16), a_hi, 0))
            bk = jnp.sum(jnp.where(lane == (k - 16), b_hi, 0))
        b2k, _ = _scan_boundary(h2s_v, k, _SAMPLES - ak)
        t22k = bk * _NBKT + b2k
        if k < 16:
            t_lo = jnp.where(lane == k, t22k, t_lo)
        else:
            t_hi = jnp.where(lane == (k - 16), t22k, t_hi)
    t22_v[pl.ds(0, 16)] = t_lo
    t22_v[pl.ds(16, 16)] = t_hi
    neg = jnp.full((16,), -3e38, jnp.float32)
    zi = jnp.zeros((16,), jnp.int32)
    for k in range(_NCLASSES):
        loc_s[pl.ds(k * _CCAP, 16)] = neg
        loc_i[pl.ds(k * _CCAP, 16)] = zi
    cnt_v[pl.ds(0, 16)] = zi
    cnt_v[pl.ds(16, 16)] = zi
    L = jnp.int32(0)
    for ch in range(_SLICE // _CHUNK):
        gbase = wid * _SLICE + ch * _CHUNK
        _load_chunk(scores_hbm, labels_hbm, sc_v, lb_v, gbase)

        def b(i, L, _gbase=gbase):
            s = sc_v[pl.ds(i * 16, 16)]
            l = lb_v[pl.ds(i * 16, 16)]
            key20 = lax.shift_right_logical(_sortable_key(s), _SH2)
            t = plsc.load_gather(t22_v, [l])
            m = key20 >= t
            cnt = jnp.sum(m.astype(jnp.int32))
            Lc = jnp.minimum(L, _LCAP - 16)
            plsc.store_compressed(cmp_s.at[pl.ds(Lc, 16)], s, mask=m)
            plsc.store_compressed(cmp_i.at[pl.ds(Lc, 16)],
                                  _gbase + i * 16 + lane, mask=m)
            plsc.store_compressed(cmp_l.at[pl.ds(Lc, 16)], l, mask=m)
            return L + cnt

        L = lax.fori_loop(0, _CHUNK // 16, b, L)
    L = jnp.minimum(L, _LCAP - 16)

    def redis(e, c):
        ev = jnp.broadcast_to(e, (16,))
        k = jnp.max(plsc.load_gather(cmp_l, [ev]))
        sc = jnp.max(plsc.load_gather(cmp_s, [ev]))
        gi = jnp.max(plsc.load_gather(cmp_i, [ev]))
        ck = jnp.max(plsc.load_gather(cnt_v, [jnp.broadcast_to(k, (16,))]))
        ckc = jnp.minimum(ck, _CCAP - 1)
        m0 = lane == 0
        pos = jnp.broadcast_to(k * _CCAP + ckc, (16,))
        plsc.store_scatter(loc_s, [pos], jnp.broadcast_to(sc, (16,)), mask=m0)
        plsc.store_scatter(loc_i, [pos], jnp.broadcast_to(gi, (16,)), mask=m0)
        plsc.store_scatter(cnt_v, [jnp.broadcast_to(k, (16,))],
                           jnp.broadcast_to(ck + 1, (16,)), mask=m0)
        return c

    lax.fori_loop(0, L, redis, 0)
    for k in range(_NCLASSES):
        pltpu.sync_copy(loc_s.at[pl.ds(k * _CCAP, _CCAP)],
                        cs_hbm.at[k, pl.ds(wid * _CCAP, _CCAP)])
        pltpu.sync_copy(loc_i.at[pl.ds(k * _CCAP, _CCAP)],
                        ci_hbm.at[k, pl.ds(wid * _CCAP, _CCAP)])


def _sc_topk(scores_flat, labels_flat):
    mesh = plsc.VectorSubcoreMesh(core_axis_name="c", subcore_axis_name="s")
    cp = pltpu.CompilerParams(needs_layout_passes=False)
    data_scratch = [
        pltpu.VMEM((_CHUNK,), jnp.float32),
        pltpu.VMEM((_CHUNK,), jnp.int32),
    ]
    h1 = pl.kernel(
        _hist1_body, mesh=mesh, compiler_params=cp,
        out_type=jax.ShapeDtypeStruct((2, _HISTW), jnp.int32),
        scratch_types=data_scratch + [
            pltpu.VMEM((_HISTW,), jnp.int32),
            pltpu.VMEM((_STRIPE,), jnp.int32),
            pltpu.VMEM((_STRIPE,), jnp.int32),
            pltpu.VMEM_SHARED((16, _HISTW), jnp.int32),
        ],
    )(scores_flat, labels_flat)
    h2, ba = pl.kernel(
        _hist2_body, mesh=mesh, compiler_params=cp,
        out_type=[jax.ShapeDtypeStruct((2, _HISTW), jnp.int32),
                  jax.ShapeDtypeStruct((64,), jnp.int32)],
        scratch_types=data_scratch + [
            pltpu.VMEM((_HISTW,), jnp.int32),
            pltpu.VMEM((_STRIPE,), jnp.int32),
            pltpu.VMEM((_STRIPE,), jnp.int32),
            pltpu.VMEM((_HISTW,), jnp.int32),
            pltpu.VMEM((32,), jnp.int32),
            pltpu.VMEM((64,), jnp.int32),
            pltpu.VMEM_SHARED((16, _HISTW), jnp.int32),
        ],
    )(scores_flat, labels_flat, h1)
    cand_s, cand_i = pl.kernel(
        _collect_body, mesh=mesh, compiler_params=cp,
        out_type=[
            jax.ShapeDtypeStruct((_NCLASSES, _NW * _CCAP), jnp.float32),
            jax.ShapeDtypeStruct((_NCLASSES, _NW * _CCAP), jnp.int32),
        ],
        scratch_types=data_scratch + [
            pltpu.VMEM((_STRIPE,), jnp.int32),
            pltpu.VMEM((_STRIPE,), jnp.int32),
            pltpu.VMEM((_HISTW,), jnp.int32),
            pltpu.VMEM((64,), jnp.int32),
            pltpu.VMEM((32,), jnp.int32),
            pltpu.VMEM((_LCAP,), jnp.float32),
            pltpu.VMEM((_LCAP,), jnp.int32),
            pltpu.VMEM((_LCAP,), jnp.int32),
            pltpu.VMEM((_NCLASSES * _CCAP,), jnp.float32),
            pltpu.VMEM((_NCLASSES * _CCAP,), jnp.int32),
            pltpu.VMEM((32,), jnp.int32),
        ],
    )(scores_flat, labels_flat, h2, ba)
    return cand_s, cand_i


_CAND = _NW * _CCAP


def _rank_kernel(sr_ref, sc_ref, ir_ref, ic_ref, out_ref):
    sj = sr_ref[0]
    si = sc_ref[0]
    ij = ir_ref[0]
    ii = ic_ref[0]
    better = jnp.logical_or(sj > si, jnp.logical_and(sj == si, ij < ii))
    rank = jnp.sum(better.astype(jnp.float32), axis=1,
                   keepdims=True).astype(jnp.int32)
    sel = (rank == jax.lax.broadcasted_iota(
        jnp.int32, (_CAND, _SAMPLES), 1)).astype(jnp.float32)
    topi = jnp.sum(sel * ic_ref[0].astype(jnp.float32), axis=0)
    out_ref[0, 0, :] = topi.astype(jnp.int32)


def _rank_sort(cand_s, cand_i):
    row = lambda a: a.reshape(_NCLASSES, 1, _CAND)
    col = lambda a: a.reshape(_NCLASSES, _CAND, 1)
    out = pl.pallas_call(
        _rank_kernel,
        grid=(_NCLASSES,),
        in_specs=[
            pl.BlockSpec((1, 1, _CAND), lambda k: (k, 0, 0)),
            pl.BlockSpec((1, _CAND, 1), lambda k: (k, 0, 0)),
            pl.BlockSpec((1, 1, _CAND), lambda k: (k, 0, 0)),
            pl.BlockSpec((1, _CAND, 1), lambda k: (k, 0, 0)),
        ],
        out_specs=pl.BlockSpec((1, 1, _SAMPLES), lambda k: (k, 0, 0)),
        out_shape=jax.ShapeDtypeStruct((_NCLASSES, 1, _SAMPLES), jnp.int32),
    )(row(cand_s), col(cand_s), row(cand_i), col(cand_i))
    return out.reshape(_NCLASSES, _SAMPLES)


def _proto_kernel(lab_ref, ft_ref, sums_ref, counts_ref):
    b = pl.program_id(0)
    j = pl.program_id(1)

    @pl.when(jnp.logical_and(b == 0, j == 0))
    def _():
        sums_ref[...] = jnp.zeros_like(sums_ref)
        counts_ref[...] = jnp.zeros_like(counts_ref)

    lab = lab_ref[0, 0, :]
    oh = (lab[:, None] == jax.lax.broadcasted_iota(jnp.int32, (_BP, _NCLASSES), 1)
          ).astype(jnp.float32)
    ft = ft_ref[0]
    sums_ref[:, :_NCLASSES] += jax.lax.dot_general(
        ft, oh, (((1,), (0,)), ((), ())))
    counts_ref[0, :_NCLASSES] += jnp.sum(oh, axis=0)


def _score_kernel(lab_ref, ft_ref, sums_ref, counts_ref, score_ref):
    proto = sums_ref[:, :_NCLASSES] / (counts_ref[0, :_NCLASSES] + 1e-6)
    ft = ft_ref[0]
    sim = jax.lax.dot_general(
        ft, proto, (((0,), (0,)), ((), ())))
    lab = lab_ref[0, 0, :]
    oh = (lab[:, None] == jax.lax.broadcasted_iota(jnp.int32, (_BP, _NCLASSES), 1)
          ).astype(jnp.float32)
    score_ref[...] = jnp.sum(sim * oh, axis=1)


def _aff_kernel(s_ref, t_ref, out_ref):
    inv = 1.0 / math.sqrt(128.0)
    out_ref[0] = jax.lax.dot_general(
        s_ref[0], t_ref[0], (((1,), (1,)), ((), ()))) * inv


def kernel(feat_s, feat_t, label_t):
    bs, c, p = feat_s.shape
    nb = p // _BP
    lab3 = label_t.reshape(bs * nb, 1, _BP)

    sums, counts = pl.pallas_call(
        _proto_kernel,
        grid=(bs, nb),
        in_specs=[
            pl.BlockSpec((1, 1, _BP), lambda b, j: (b * nb + j, 0, 0)),
            pl.BlockSpec((1, c, _BP), lambda b, j: (b, 0, j)),
        ],
        out_specs=[
            pl.BlockSpec((c, 32), lambda b, j: (0, 0)),
            pl.BlockSpec((8, 32), lambda b, j: (0, 0)),
        ],
        out_shape=[
            jax.ShapeDtypeStruct((c, 32), jnp.float32),
            jax.ShapeDtypeStruct((8, 32), jnp.float32),
        ],
    )(lab3, feat_t)

    scores = pl.pallas_call(
        _score_kernel,
        grid=(bs, nb),
        in_specs=[
            pl.BlockSpec((1, 1, _BP), lambda b, j: (b * nb + j, 0, 0)),
            pl.BlockSpec((1, c, _BP), lambda b, j: (b, 0, j)),
            pl.BlockSpec((c, 32), lambda b, j: (0, 0)),
            pl.BlockSpec((8, 32), lambda b, j: (0, 0)),
        ],
        out_specs=pl.BlockSpec((_BP,), lambda b, j: (b * nb + j,)),
        out_shape=jax.ShapeDtypeStruct((bs * nb * _BP,), jnp.float32),
    )(lab3, feat_t, sums, counts)

    cand_s, cand_i = _sc_topk(scores, label_t.reshape(-1))
    topi = _rank_sort(cand_s, cand_i)

    samp_s, samp_t = _sc_gather(
        topi.reshape(-1),
        feat_s.reshape(bs * c * p),
        feat_t.reshape(bs * c * p))
    sampled_s = samp_s.reshape(_NCLASSES, _SAMPLES, c)
    sampled_t = samp_t.reshape(_NCLASSES, _SAMPLES, c)

    aff = pl.pallas_call(
        _aff_kernel,
        grid=(_NCLASSES,),
        in_specs=[
            pl.BlockSpec((1, _SAMPLES, c), lambda k: (k, 0, 0)),
            pl.BlockSpec((1, _SAMPLES, c), lambda k: (k, 0, 0)),
        ],
        out_specs=pl.BlockSpec((1, _SAMPLES, _SAMPLES), lambda k: (k, 0, 0)),
        out_shape=jax.ShapeDtypeStruct((_NCLASSES, _SAMPLES, _SAMPLES), jnp.float32),
    )(sampled_s, sampled_t)
    return aff


# speedup vs baseline: 1.3108x; 1.3108x over previous
import functools
import math

import jax
import jax.numpy as jnp
from jax import lax
from jax.experimental import pallas as pl
from jax.experimental.pallas import tpu as pltpu
from jax.experimental.pallas import tpu_sc as plsc

_NCLASSES = 20
_SAMPLES = 128
_BP = 2048
_NW = 32
_GPW = (_NCLASSES * _SAMPLES) // _NW
_PSTRIDE = 131072
_CH = 128


def _gather_body(topi_hbm, fs_hbm, ft_hbm, out_s, out_t,
                 idx_v, bases_v, chidx_v, rows_s, rows_t, sem_s, sem_t):
    wid = lax.axis_index("s") * 2 + lax.axis_index("c")
    base = wid * _GPW
    pltpu.sync_copy(topi_hbm.at[pl.ds(base, _GPW)], idx_v)
    lane = lax.iota(jnp.int32, 16)
    for t in range(_GPW // 16):
        nvec = idx_v[pl.ds(t * 16, 16)]
        bb = lax.shift_right_logical(nvec, 17)
        ii = jnp.bitwise_and(nvec, jnp.int32(_PSTRIDE - 1))
        bvec = (bb * (_CH * _PSTRIDE)
                + lax.shift_right_logical(ii, 7) * 1024
                + jnp.bitwise_and(ii, jnp.int32(127)))
        for b in range(16):
            j = t * 16 + b
            bj = jnp.broadcast_to(
                jnp.sum(jnp.where(lane == b, bvec, 0)), (16,))
            for g in range(8):
                cv = lane + g * 16
                chidx_v[j, pl.ds(g * 16, 16)] = (
                    bj + lax.shift_right_logical(cv, 3) * (_PSTRIDE * 8)
                    + jnp.bitwise_and(cv, jnp.int32(7)) * 128)
    for j in range(_GPW):
        pltpu.async_copy(fs_hbm.at[chidx_v.at[j]], rows_s.at[j], sem_s)
        pltpu.async_copy(ft_hbm.at[chidx_v.at[j]], rows_t.at[j], sem_t)
    for j in range(_GPW):
        pltpu.make_async_copy(fs_hbm.at[chidx_v.at[j]], rows_s.at[j],
                              sem_s).wait()
        pltpu.make_async_copy(ft_hbm.at[chidx_v.at[j]], rows_t.at[j],
                              sem_t).wait()
    pltpu.sync_copy(rows_s, out_s.at[pl.ds(base, _GPW)])
    pltpu.sync_copy(rows_t, out_t.at[pl.ds(base, _GPW)])


def _sc_gather(topi_flat, fs_flat, ft_flat):
    n_samp = _NCLASSES * _SAMPLES
    mesh = plsc.VectorSubcoreMesh(core_axis_name="c", subcore_axis_name="s")
    f = pl.kernel(
        _gather_body,
        mesh=mesh,
        compiler_params=pltpu.CompilerParams(needs_layout_passes=False),
        out_type=[
            jax.ShapeDtypeStruct((n_samp, _CH), jnp.float32),
            jax.ShapeDtypeStruct((n_samp, _CH), jnp.float32),
        ],
        scratch_types=[
            pltpu.VMEM((_GPW,), jnp.int32),
            pltpu.VMEM((_GPW,), jnp.int32),
            pltpu.VMEM((_GPW, _CH), jnp.int32),
            pltpu.VMEM((_GPW, _CH), jnp.float32),
            pltpu.VMEM((_GPW, _CH), jnp.float32),
            pltpu.SemaphoreType.DMA,
            pltpu.SemaphoreType.DMA,
        ],
    )
    return f(topi_flat, fs_flat, ft_flat)


_NBKT = 1024
_HISTW = _NCLASSES * _NBKT
_SLICE = 262144 // _NW
_CHUNK = 2048
_STRIPE = _HISTW // 16
_SH1 = 32 - 10
_SH2 = 32 - 20
_CCAP = 16
_LCAP = 240


def _sortable_key(s):
    k1 = plsc.bitcast(s, jnp.int32)
    return jnp.where(k1 < 0, ~k1, k1 | jnp.int32(-(2 ** 31)))


def _load_chunk(scores_hbm, labels_hbm, sc_v, lb_v, base):
    pltpu.sync_copy(scores_hbm.at[pl.ds(base, _CHUNK)], sc_v)
    pltpu.sync_copy(labels_hbm.at[pl.ds(base, _CHUNK)], lb_v)


def _zero_vmem_i32(ref, nwords):
    z = jnp.zeros((16,), jnp.int32)

    def b(i, c):
        ref[pl.ds(i * 16, 16)] = z
        return c

    lax.fori_loop(0, nwords // 16, b, 0)


def _publish_reduce(hist_v, spmem, red_v, tmp_v, sid, cid, out_hbm):
    pltpu.sync_copy(hist_v, spmem.at[sid])
    plsc.subcore_barrier()
    pltpu.sync_copy(spmem.at[0, pl.ds(sid * _STRIPE, _STRIPE)], red_v)
    for r in range(1, 16):
        pltpu.sync_copy(spmem.at[r, pl.ds(sid * _STRIPE, _STRIPE)], tmp_v)

        def badd(i, c):
            red_v[pl.ds(i * 16, 16)] = (red_v[pl.ds(i * 16, 16)]
                                        + tmp_v[pl.ds(i * 16, 16)])
            return c

        lax.fori_loop(0, _STRIPE // 16, badd, 0)
    pltpu.sync_copy(red_v, out_hbm.at[cid, pl.ds(sid * _STRIPE, _STRIPE)])


def _scan_boundary(hist_ref, k, target):
    lane = lax.iota(jnp.int32, 16)

    def cond(st):
        return jnp.logical_not(st[4])

    def body(st):
        v, cum, bb, aa, done = st
        hv = hist_ref[pl.ds(k * _NBKT + v * 16, 16)]
        rc = lax.rev(hv, (0,))
        cs = jnp.cumsum(rc)
        cum_incl = cum + cs
        mask = cum_incl >= target
        mask = jnp.logical_or(mask, jnp.logical_and(v == 0, lane == 15))
        anyhit = jnp.max(mask.astype(jnp.int32)) > 0
        f = jnp.max(plsc.all_reduce_ffs(mask))
        bnew = v * 16 + 15 - f
        csm1 = jnp.sum(jnp.where(lane == f - 1, cs, 0))
        anew = cum + csm1
        vec_total = jnp.sum(hv)
        return (jnp.where(anyhit, v, v - 1),
                jnp.where(anyhit, cum, cum + vec_total),
                jnp.where(anyhit, bnew, bb),
                jnp.where(anyhit, anew, aa),
                anyhit)

    st = lax.while_loop(cond, body, (jnp.int32(_NBKT // 16 - 1),
                                     jnp.int32(0), jnp.int32(0),
                                     jnp.int32(0), False))
    return st[2], st[3]


def _hist1_body(scores_hbm, labels_hbm, out_hbm,
                sc_v, lb_v, hist_v, red_v, tmp_v, spmem):
    cid = lax.axis_index("c")
    sid = lax.axis_index("s")
    wid = cid * 16 + sid
    _zero_vmem_i32(hist_v, _HISTW)
    ones = jnp.ones((16,), jnp.int32)
    for ch in range(_SLICE // _CHUNK):
        _load_chunk(scores_hbm, labels_hbm, sc_v, lb_v,
                    wid * _SLICE + ch * _CHUNK)

        def b(i, c):
            s = sc_v[pl.ds(i * 16, 16)]
            l = lb_v[pl.ds(i * 16, 16)]
            key = _sortable_key(s)
            b1 = lax.shift_right_logical(key, _SH1)
            plsc.addupdate_scatter(hist_v, [l * _NBKT + b1], ones)
            return c

        lax.fori_loop(0, _CHUNK // 16, b, 0)
    _publish_reduce(hist_v, spmem, red_v, tmp_v, sid, cid, out_hbm)


def _hist2_body(scores_hbm, labels_hbm, h1_hbm, out_hbm, ba_hbm,
                sc_v, lb_v, hist_v, red_v, tmp_v, h1s_v, btab_v, ba_v, spmem):
    cid = lax.axis_index("c")
    sid = lax.axis_index("s")
    wid = cid * 16 + sid
    lane = lax.iota(jnp.int32, 16)
    for part in range(16):
        pltpu.sync_copy(h1_hbm.at[0, pl.ds(part * _STRIPE, _STRIPE)], red_v)
        pltpu.sync_copy(h1_hbm.at[1, pl.ds(part * _STRIPE, _STRIPE)], tmp_v)

        def badd(i, c, _part=part):
            h1s_v[pl.ds(_part * _STRIPE + i * 16, 16)] = (
                red_v[pl.ds(i * 16, 16)] + tmp_v[pl.ds(i * 16, 16)])
            return c

        lax.fori_loop(0, _STRIPE // 16, badd, 0)
    b_lo = jnp.zeros((16,), jnp.int32)
    b_hi = jnp.zeros((16,), jnp.int32)
    a_lo = jnp.zeros((16,), jnp.int32)
    a_hi = jnp.zeros((16,), jnp.int32)
    for k in range(_NCLASSES):
        bk, ak = _scan_boundary(h1s_v, k, _SAMPLES)
        if k < 16:
            b_lo = jnp.where(lane == k, bk, b_lo)
            a_lo = jnp.where(lane == k, ak, a_lo)
        else:
            b_hi = jnp.where(lane == (k - 16), bk, b_hi)
            a_hi = jnp.where(lane == (k - 16), ak, a_hi)
    btab_v[pl.ds(0, 16)] = b_lo
    btab_v[pl.ds(16, 16)] = b_hi
    ba_v[pl.ds(0, 16)] = b_lo
    ba_v[pl.ds(16, 16)] = b_hi
    ba_v[pl.ds(32, 16)] = a_lo
    ba_v[pl.ds(48, 16)] = a_hi

    @pl.when(wid == 0)
    def _():
        pltpu.sync_copy(ba_v, ba_hbm)

    _zero_vmem_i32(hist_v, _HISTW)
    ones = jnp.ones((16,), jnp.int32)
    for ch in range(_SLICE // _CHUNK):
        _load_chunk(scores_hbm, labels_hbm, sc_v, lb_v,
                    wid * _SLICE + ch * _CHUNK)

        def b(i, c):
            s = sc_v[pl.ds(i * 16, 16)]
            l = lb_v[pl.ds(i * 16, 16)]
            key = _sortable_key(s)
            b1 = lax.shift_right_logical(key, _SH1)
            sub = jnp.bitwise_and(lax.shift_right_logical(key, _SH2),
                                  jnp.int32(_NBKT - 1))
            bl = plsc.load_gather(btab_v, [l])
            m = b1 == bl
            plsc.addupdate_scatter(hist_v, [l * _NBKT + sub], ones, mask=m)
            return c

        lax.fori_loop(0, _CHUNK // 16, b, 0)
    _publish_reduce(hist_v, spmem, red_v, tmp_v, sid, cid, out_hbm)


def _collect_body(scores_hbm, labels_hbm, h2_hbm, ba_hbm,
                  cs_hbm, ci_hbm,
                  sc_v, lb_v, red_v, tmp_v, h2s_v, ba_v, t22_v,
                  cmp_s, cmp_i, cmp_l, loc_s, loc_i, cnt_v):
    cid = lax.axis_index("c")
    sid = lax.axis_index("s")
    wid = cid * 16 + sid
    lane = lax.iota(jnp.int32, 16)
    for part in range(16):
        pltpu.sync_copy(h2_hbm.at[0, pl.ds(part * _STRIPE, _STRIPE)], red_v)
        pltpu.sync_copy(h2_hbm.at[1, pl.ds(part * _STRIPE, _STRIPE)], tmp_v)

        def badd(i, c, _part=part):
            h2s_v[pl.ds(_part * _STRIPE + i * 16, 16)] = (
                red_v[pl.ds(i * 16, 16)] + tmp_v[pl.ds(i * 16, 16)])
            return c

        lax.fori_loop(0, _STRIPE // 16, badd, 0)
    pltpu.sync_copy(ba_hbm, ba_v)
    b_lo = ba_v[pl.ds(0, 16)]
    b_hi = ba_v[pl.ds(16, 16)]
    a_lo = ba_v[pl.ds(32, 16)]
    a_hi = ba_v[pl.ds(48, 16)]
    t_lo = jnp.zeros((16,), jnp.int32)
    t_hi = jnp.zeros((16,), jnp.int32)
    for k in range(_NCLASSES):
        if k < 16:
            ak = jnp.sum(jnp.where(lane == k, a_lo, 0))
            bk = jnp.sum(jnp.where(lane == k, b_lo, 0))
        else:
            ak = jnp.sum(jnp.where(lane == (k - 16), a_hi, 0))
            bk = jnp.sum(jnp.where(lane == (k - 16), b_hi, 0))
        b2k, _ = _scan_boundary(h2s_v, k, _SAMPLES - ak)
        t22k = bk * _NBKT + b2k
        if k < 16:
            t_lo = jnp.where(lane == k, t22k, t_lo)
        else:
            t_hi = jnp.where(lane == (k - 16), t22k, t_hi)
    t22_v[pl.ds(0, 16)] = t_lo
    t22_v[pl.ds(16, 16)] = t_hi
    neg = jnp.full((16,), -3e38, jnp.float32)
    zi = jnp.zeros((16,), jnp.int32)
    for k in range(_NCLASSES):
        loc_s[pl.ds(k * _CCAP, 16)] = neg
        loc_i[pl.ds(k * _CCAP, 16)] = zi
    cnt_v[pl.ds(0, 16)] = zi
    cnt_v[pl.ds(16, 16)] = zi
    L = jnp.int32(0)
    for ch in range(_SLICE // _CHUNK):
        gbase = wid * _SLICE + ch * _CHUNK
        _load_chunk(scores_hbm, labels_hbm, sc_v, lb_v, gbase)

        def b(i, L, _gbase=gbase):
            s = sc_v[pl.ds(i * 16, 16)]
            l = lb_v[pl.ds(i * 16, 16)]
            key20 = lax.shift_right_logical(_sortable_key(s), _SH2)
            t = plsc.load_gather(t22_v, [l])
            m = key20 >= t
            cnt = jnp.sum(m.astype(jnp.int32))
            Lc = jnp.minimum(L, _LCAP - 16)
            plsc.store_compressed(cmp_s.at[pl.ds(Lc, 16)], s, mask=m)
            plsc.store_compressed(cmp_i.at[pl.ds(Lc, 16)],
                                  _gbase + i * 16 + lane, mask=m)
            plsc.store_compressed(cmp_l.at[pl.ds(Lc, 16)], l, mask=m)
            return L + cnt

        L = lax.fori_loop(0, _CHUNK // 16, b, L)
    L = jnp.minimum(L, _LCAP - 16)

    def redis(e, c):
        ev = jnp.broadcast_to(e, (16,))
        k = jnp.max(plsc.load_gather(cmp_l, [ev]))
        sc = jnp.max(plsc.load_gather(cmp_s, [ev]))
        gi = jnp.max(plsc.load_gather(cmp_i, [ev]))
        ck = jnp.max(plsc.load_gather(cnt_v, [jnp.broadcast_to(k, (16,))]))
        ckc = jnp.minimum(ck, _CCAP - 1)
        m0 = lane == 0
        pos = jnp.broadcast_to(k * _CCAP + ckc, (16,))
        plsc.store_scatter(loc_s, [pos], jnp.broadcast_to(sc, (16,)), mask=m0)
        plsc.store_scatter(loc_i, [pos], jnp.broadcast_to(gi, (16,)), mask=m0)
        plsc.store_scatter(cnt_v, [jnp.broadcast_to(k, (16,))],
                           jnp.broadcast_to(ck + 1, (16,)), mask=m0)
        return c

    lax.fori_loop(0, L, redis, 0)
    for k in range(_NCLASSES):
        pltpu.sync_copy(loc_s.at[pl.ds(k * _CCAP, _CCAP)],
                        cs_hbm.at[k, pl.ds(wid * _CCAP, _CCAP)])
        pltpu.sync_copy(loc_i.at[pl.ds(k * _CCAP, _CCAP)],
                        ci_hbm.at[k, pl.ds(wid * _CCAP, _CCAP)])


def _sc_topk(scores_flat, labels_flat):
    mesh = plsc.VectorSubcoreMesh(core_axis_name="c", subcore_axis_name="s")
    cp = pltpu.CompilerParams(needs_layout_passes=False)
    data_scratch = [
        pltpu.VMEM((_CHUNK,), jnp.float32),
        pltpu.VMEM((_CHUNK,), jnp.int32),
    ]
    h1 = pl.kernel(
        _hist1_body, mesh=mesh, compiler_params=cp,
        out_type=jax.ShapeDtypeStruct((2, _HISTW), jnp.int32),
        scratch_types=data_scratch + [
            pltpu.VMEM((_HISTW,), jnp.int32),
            pltpu.VMEM((_STRIPE,), jnp.int32),
            pltpu.VMEM((_STRIPE,), jnp.int32),
            pltpu.VMEM_SHARED((16, _HISTW), jnp.int32),
        ],
    )(scores_flat, labels_flat)
    h2, ba = pl.kernel(
        _hist2_body, mesh=mesh, compiler_params=cp,
        out_type=[jax.ShapeDtypeStruct((2, _HISTW), jnp.int32),
                  jax.ShapeDtypeStruct((64,), jnp.int32)],
        scratch_types=data_scratch + [
            pltpu.VMEM((_HISTW,), jnp.int32),
            pltpu.VMEM((_STRIPE,), jnp.int32),
            pltpu.VMEM((_STRIPE,), jnp.int32),
            pltpu.VMEM((_HISTW,), jnp.int32),
            pltpu.VMEM((32,), jnp.int32),
            pltpu.VMEM((64,), jnp.int32),
            pltpu.VMEM_SHARED((16, _HISTW), jnp.int32),
        ],
    )(scores_flat, labels_flat, h1)
    cand_s, cand_i = pl.kernel(
        _collect_body, mesh=mesh, compiler_params=cp,
        out_type=[
            jax.ShapeDtypeStruct((_NCLASSES, _NW * _CCAP), jnp.float32),
            jax.ShapeDtypeStruct((_NCLASSES, _NW * _CCAP), jnp.int32),
        ],
        scratch_types=data_scratch + [
            pltpu.VMEM((_STRIPE,), jnp.int32),
            pltpu.VMEM((_STRIPE,), jnp.int32),
            pltpu.VMEM((_HISTW,), jnp.int32),
            pltpu.VMEM((64,), jnp.int32),
            pltpu.VMEM((32,), jnp.int32),
            pltpu.VMEM((_LCAP,), jnp.float32),
            pltpu.VMEM((_LCAP,), jnp.int32),
            pltpu.VMEM((_LCAP,), jnp.int32),
            pltpu.VMEM((_NCLASSES * _CCAP,), jnp.float32),
            pltpu.VMEM((_NCLASSES * _CCAP,), jnp.int32),
            pltpu.VMEM((32,), jnp.int32),
        ],
    )(scores_flat, labels_flat, h2, ba)
    return cand_s, cand_i


_CAND = _NW * _CCAP


def _rank_kernel(sr_ref, sc_ref, ir_ref, ic_ref, out_ref):
    sj = sr_ref[0]
    si = sc_ref[0]
    ij = ir_ref[0]
    ii = ic_ref[0]
    better = jnp.logical_or(sj > si, jnp.logical_and(sj == si, ij < ii))
    rank = jnp.sum(better.astype(jnp.float32), axis=1,
                   keepdims=True).astype(jnp.int32)
    sel = (rank == jax.lax.broadcasted_iota(
        jnp.int32, (_CAND, _SAMPLES), 1)).astype(jnp.float32)
    topi = jnp.sum(sel * ic_ref[0].astype(jnp.float32), axis=0)
    out_ref[0, 0, :] = topi.astype(jnp.int32)


def _rank_sort(cand_s, cand_i):
    row = lambda a: a.reshape(_NCLASSES, 1, _CAND)
    col = lambda a: a.reshape(_NCLASSES, _CAND, 1)
    out = pl.pallas_call(
        _rank_kernel,
        grid=(_NCLASSES,),
        in_specs=[
            pl.BlockSpec((1, 1, _CAND), lambda k: (k, 0, 0)),
            pl.BlockSpec((1, _CAND, 1), lambda k: (k, 0, 0)),
            pl.BlockSpec((1, 1, _CAND), lambda k: (k, 0, 0)),
            pl.BlockSpec((1, _CAND, 1), lambda k: (k, 0, 0)),
        ],
        out_specs=pl.BlockSpec((1, 1, _SAMPLES), lambda k: (k, 0, 0)),
        out_shape=jax.ShapeDtypeStruct((_NCLASSES, 1, _SAMPLES), jnp.int32),
    )(row(cand_s), col(cand_s), row(cand_i), col(cand_i))
    return out.reshape(_NCLASSES, _SAMPLES)


def _proto_kernel(lab_ref, ft_ref, sums_ref, counts_ref):
    b = pl.program_id(0)
    j = pl.program_id(1)

    @pl.when(jnp.logical_and(b == 0, j == 0))
    def _():
        sums_ref[...] = jnp.zeros_like(sums_ref)
        counts_ref[...] = jnp.zeros_like(counts_ref)

    lab = lab_ref[0, 0, :]
    oh = (lab[:, None] == jax.lax.broadcasted_iota(jnp.int32, (_BP, _NCLASSES), 1)
          ).astype(jnp.float32)
    ft = ft_ref[0]
    sums_ref[:, :_NCLASSES] += jax.lax.dot_general(
        ft, oh, (((1,), (0,)), ((), ())))
    counts_ref[0, :_NCLASSES] += jnp.sum(oh, axis=0)


def _score_kernel(lab_ref, ft_ref, sums_ref, counts_ref, score_ref):
    proto = sums_ref[:, :_NCLASSES] / (counts_ref[0, :_NCLASSES] + 1e-6)
    ft = ft_ref[0]
    sim = jax.lax.dot_general(
        ft, proto, (((0,), (0,)), ((), ())))
    lab = lab_ref[0, 0, :]
    oh = (lab[:, None] == jax.lax.broadcasted_iota(jnp.int32, (_BP, _NCLASSES), 1)
          ).astype(jnp.float32)
    score_ref[...] = jnp.sum(sim * oh, axis=1)


def _aff_kernel(s_ref, t_ref, out_ref):
    inv = 1.0 / math.sqrt(128.0)
    out_ref[0] = jax.lax.dot_general(
        s_ref[0], t_ref[0], (((1,), (1,)), ((), ()))) * inv


def kernel(feat_s, feat_t, label_t):
    bs, c, p = feat_s.shape
    nb = p // _BP
    lab3 = label_t.reshape(bs * nb, 1, _BP)

    sums, counts = pl.pallas_call(
        _proto_kernel,
        grid=(bs, nb),
        in_specs=[
            pl.BlockSpec((1, 1, _BP), lambda b, j: (b * nb + j, 0, 0)),
            pl.BlockSpec((1, c, _BP), lambda b, j: (b, 0, j)),
        ],
        out_specs=[
            pl.BlockSpec((c, 32), lambda b, j: (0, 0)),
            pl.BlockSpec((8, 32), lambda b, j: (0, 0)),
        ],
        out_shape=[
            jax.ShapeDtypeStruct((c, 32), jnp.float32),
            jax.ShapeDtypeStruct((8, 32), jnp.float32),
        ],
    )(lab3, feat_t)

    scores = pl.pallas_call(
        _score_kernel,
        grid=(bs, nb),
        in_specs=[
            pl.BlockSpec((1, 1, _BP), lambda b, j: (b * nb + j, 0, 0)),
            pl.BlockSpec((1, c, _BP), lambda b, j: (b, 0, j)),
            pl.BlockSpec((c, 32), lambda b, j: (0, 0)),
            pl.BlockSpec((8, 32), lambda b, j: (0, 0)),
        ],
        out_specs=pl.BlockSpec((_BP,), lambda b, j: (b * nb + j,)),
        out_shape=jax.ShapeDtypeStruct((bs * nb * _BP,), jnp.float32),
    )(lab3, feat_t, sums, counts)

    cand_s, cand_i = _sc_topk(scores, label_t.reshape(-1))
    topi = _rank_sort(cand_s, cand_i)

    def _physical_view(x):
        return x.reshape(bs, c // 8, 8, p // 128, 128).transpose(
            0, 1, 3, 2, 4).reshape(bs * c * p)

    samp_s, samp_t = _sc_gather(
        topi.reshape(-1), _physical_view(feat_s), _physical_view(feat_t))
    sampled_s = samp_s.reshape(_NCLASSES, _SAMPLES, c)
    sampled_t = samp_t.reshape(_NCLASSES, _SAMPLES, c)

    aff = pl.pallas_call(
        _aff_kernel,
        grid=(_NCLASSES,),
        in_specs=[
            pl.BlockSpec((1, _SAMPLES, c), lambda k: (k, 0, 0)),
            pl.BlockSpec((1, _SAMPLES, c), lambda k: (k, 0, 0)),
        ],
        out_specs=pl.BlockSpec((1, _SAMPLES, _SAMPLES), lambda k: (k, 0, 0)),
        out_shape=jax.ShapeDtypeStruct((_NCLASSES, _SAMPLES, _SAMPLES), jnp.float32),
    )(sampled_s, sampled_t)
    return aff


# speedup vs baseline: 1.5853x; 1.2093x over previous
import functools
import math

import jax
import jax.numpy as jnp
from jax import lax
from jax.experimental import pallas as pl
from jax.experimental.pallas import tpu as pltpu
from jax.experimental.pallas import tpu_sc as plsc

_NCLASSES = 20
_SAMPLES = 128
_BP = 2048
_NW = 32
_GPW = (_NCLASSES * _SAMPLES) // _NW
_PSTRIDE = 131072
_CH = 128


def _gather_body(topi_hbm, fs_hbm, ft_hbm, out_s, out_t,
                 idx_v, bases_v, chidx_v, rows_s, rows_t, sem_s, sem_t):
    wid = lax.axis_index("s") * 2 + lax.axis_index("c")
    base = wid * _GPW
    pltpu.sync_copy(topi_hbm.at[pl.ds(base, _GPW)], idx_v)
    lane = lax.iota(jnp.int32, 16)
    for t in range(_GPW // 16):
        nvec = idx_v[pl.ds(t * 16, 16)]
        bb = lax.shift_right_logical(nvec, 17)
        ii = jnp.bitwise_and(nvec, jnp.int32(_PSTRIDE - 1))
        bvec = (bb * (_CH * _PSTRIDE)
                + lax.shift_right_logical(ii, 7) * 1024
                + jnp.bitwise_and(ii, jnp.int32(127)))
        for b in range(16):
            j = t * 16 + b
            bj = jnp.broadcast_to(
                jnp.sum(jnp.where(lane == b, bvec, 0)), (16,))
            for g in range(8):
                cv = lane + g * 16
                chidx_v[j, pl.ds(g * 16, 16)] = (
                    bj + lax.shift_right_logical(cv, 3) * (_PSTRIDE * 8)
                    + jnp.bitwise_and(cv, jnp.int32(7)) * 128)
    for j in range(_GPW):
        pltpu.async_copy(fs_hbm.at[chidx_v.at[j]], rows_s.at[j], sem_s)
        pltpu.async_copy(ft_hbm.at[chidx_v.at[j]], rows_t.at[j], sem_t)
    for j in range(_GPW):
        pltpu.make_async_copy(fs_hbm.at[chidx_v.at[j]], rows_s.at[j],
                              sem_s).wait()
        pltpu.make_async_copy(ft_hbm.at[chidx_v.at[j]], rows_t.at[j],
                              sem_t).wait()
    pltpu.sync_copy(rows_s, out_s.at[pl.ds(base, _GPW)])
    pltpu.sync_copy(rows_t, out_t.at[pl.ds(base, _GPW)])


def _sc_gather(topi_flat, fs_flat, ft_flat):
    n_samp = _NCLASSES * _SAMPLES
    mesh = plsc.VectorSubcoreMesh(core_axis_name="c", subcore_axis_name="s")
    f = pl.kernel(
        _gather_body,
        mesh=mesh,
        compiler_params=pltpu.CompilerParams(needs_layout_passes=False),
        out_type=[
            jax.ShapeDtypeStruct((n_samp, _CH), jnp.float32),
            jax.ShapeDtypeStruct((n_samp, _CH), jnp.float32),
        ],
        scratch_types=[
            pltpu.VMEM((_GPW,), jnp.int32),
            pltpu.VMEM((_GPW,), jnp.int32),
            pltpu.VMEM((_GPW, _CH), jnp.int32),
            pltpu.VMEM((_GPW, _CH), jnp.float32),
            pltpu.VMEM((_GPW, _CH), jnp.float32),
            pltpu.SemaphoreType.DMA,
            pltpu.SemaphoreType.DMA,
        ],
    )
    return f(topi_flat, fs_flat, ft_flat)


_NBKT = 1024
_HISTW = _NCLASSES * _NBKT
_SLICE = 262144 // _NW
_CHUNK = 2048
_STRIPE = _HISTW // 16
_SH1 = 32 - 10
_SH2 = 32 - 20
_CCAP = 16
_LCAP = 240


def _sortable_key(s):
    k1 = plsc.bitcast(s, jnp.int32)
    return jnp.where(k1 < 0, ~k1, k1 | jnp.int32(-(2 ** 31)))


def _load_chunk(scores_hbm, labels_hbm, sc_v, lb_v, base):
    pltpu.sync_copy(scores_hbm.at[pl.ds(base, _CHUNK)], sc_v)
    pltpu.sync_copy(labels_hbm.at[pl.ds(base, _CHUNK)], lb_v)


def _zero_vmem_i32(ref, nwords):
    z = jnp.zeros((16,), jnp.int32)

    def b(i, c):
        ref[pl.ds(i * 16, 16)] = z
        return c

    lax.fori_loop(0, nwords // 16, b, 0)


def _publish_reduce(hist_v, spmem, red_v, tmp_v, sid, cid, out_hbm):
    pltpu.sync_copy(hist_v, spmem.at[sid])
    plsc.subcore_barrier()
    pltpu.sync_copy(spmem.at[0, pl.ds(sid * _STRIPE, _STRIPE)], red_v)
    for r in range(1, 16):
        pltpu.sync_copy(spmem.at[r, pl.ds(sid * _STRIPE, _STRIPE)], tmp_v)

        def badd(i, c):
            red_v[pl.ds(i * 16, 16)] = (red_v[pl.ds(i * 16, 16)]
                                        + tmp_v[pl.ds(i * 16, 16)])
            return c

        lax.fori_loop(0, _STRIPE // 16, badd, 0)
    pltpu.sync_copy(red_v, out_hbm.at[cid, pl.ds(sid * _STRIPE, _STRIPE)])


def _scan_boundary(hist_ref, k, target):
    lane = lax.iota(jnp.int32, 16)

    def cond(st):
        return jnp.logical_not(st[4])

    def body(st):
        v, cum, bb, aa, done = st
        hv = hist_ref[pl.ds(k * _NBKT + v * 16, 16)]
        rc = lax.rev(hv, (0,))
        cs = jnp.cumsum(rc)
        cum_incl = cum + cs
        mask = cum_incl >= target
        mask = jnp.logical_or(mask, jnp.logical_and(v == 0, lane == 15))
        anyhit = jnp.max(mask.astype(jnp.int32)) > 0
        f = jnp.max(plsc.all_reduce_ffs(mask))
        bnew = v * 16 + 15 - f
        csm1 = jnp.sum(jnp.where(lane == f - 1, cs, 0))
        anew = cum + csm1
        vec_total = jnp.sum(hv)
        return (jnp.where(anyhit, v, v - 1),
                jnp.where(anyhit, cum, cum + vec_total),
                jnp.where(anyhit, bnew, bb),
                jnp.where(anyhit, anew, aa),
                anyhit)

    st = lax.while_loop(cond, body, (jnp.int32(_NBKT // 16 - 1),
                                     jnp.int32(0), jnp.int32(0),
                                     jnp.int32(0), False))
    return st[2], st[3]


def _hist1_body(scores_hbm, labels_hbm, out_hbm,
                sc_v, lb_v, hist_v, red_v, tmp_v, spmem):
    cid = lax.axis_index("c")
    sid = lax.axis_index("s")
    wid = cid * 16 + sid
    _zero_vmem_i32(hist_v, _HISTW)
    ones = jnp.ones((16,), jnp.int32)
    for ch in range(_SLICE // _CHUNK):
        _load_chunk(scores_hbm, labels_hbm, sc_v, lb_v,
                    wid * _SLICE + ch * _CHUNK)

        def b(i, c):
            s = sc_v[pl.ds(i * 16, 16)]
            l = lb_v[pl.ds(i * 16, 16)]
            key = _sortable_key(s)
            b1 = lax.shift_right_logical(key, _SH1)
            plsc.addupdate_scatter(hist_v, [l * _NBKT + b1], ones)
            return c

        lax.fori_loop(0, _CHUNK // 16, b, 0)
    _publish_reduce(hist_v, spmem, red_v, tmp_v, sid, cid, out_hbm)


def _hist2_body(scores_hbm, labels_hbm, h1_hbm, out_hbm, ba_hbm,
                sc_v, lb_v, hist_v, red_v, tmp_v, h1s_v, btab_v, ba_v, spmem):
    cid = lax.axis_index("c")
    sid = lax.axis_index("s")
    wid = cid * 16 + sid
    lane = lax.iota(jnp.int32, 16)
    for part in range(16):
        pltpu.sync_copy(h1_hbm.at[0, pl.ds(part * _STRIPE, _STRIPE)], red_v)
        pltpu.sync_copy(h1_hbm.at[1, pl.ds(part * _STRIPE, _STRIPE)], tmp_v)

        def badd(i, c, _part=part):
            h1s_v[pl.ds(_part * _STRIPE + i * 16, 16)] = (
                red_v[pl.ds(i * 16, 16)] + tmp_v[pl.ds(i * 16, 16)])
            return c

        lax.fori_loop(0, _STRIPE // 16, badd, 0)
    b_lo = jnp.zeros((16,), jnp.int32)
    b_hi = jnp.zeros((16,), jnp.int32)
    a_lo = jnp.zeros((16,), jnp.int32)
    a_hi = jnp.zeros((16,), jnp.int32)
    for k in range(_NCLASSES):
        bk, ak = _scan_boundary(h1s_v, k, _SAMPLES)
        if k < 16:
            b_lo = jnp.where(lane == k, bk, b_lo)
            a_lo = jnp.where(lane == k, ak, a_lo)
        else:
            b_hi = jnp.where(lane == (k - 16), bk, b_hi)
            a_hi = jnp.where(lane == (k - 16), ak, a_hi)
    btab_v[pl.ds(0, 16)] = b_lo
    btab_v[pl.ds(16, 16)] = b_hi
    ba_v[pl.ds(0, 16)] = b_lo
    ba_v[pl.ds(16, 16)] = b_hi
    ba_v[pl.ds(32, 16)] = a_lo
    ba_v[pl.ds(48, 16)] = a_hi

    @pl.when(wid == 0)
    def _():
        pltpu.sync_copy(ba_v, ba_hbm)

    _zero_vmem_i32(hist_v, _HISTW)
    ones = jnp.ones((16,), jnp.int32)
    for ch in range(_SLICE // _CHUNK):
        _load_chunk(scores_hbm, labels_hbm, sc_v, lb_v,
                    wid * _SLICE + ch * _CHUNK)

        def b(i, c):
            s = sc_v[pl.ds(i * 16, 16)]
            l = lb_v[pl.ds(i * 16, 16)]
            key = _sortable_key(s)
            b1 = lax.shift_right_logical(key, _SH1)
            sub = jnp.bitwise_and(lax.shift_right_logical(key, _SH2),
                                  jnp.int32(_NBKT - 1))
            bl = plsc.load_gather(btab_v, [l])
            m = b1 == bl
            plsc.addupdate_scatter(hist_v, [l * _NBKT + sub], ones, mask=m)
            return c

        lax.fori_loop(0, _CHUNK // 16, b, 0)
    _publish_reduce(hist_v, spmem, red_v, tmp_v, sid, cid, out_hbm)


def _collect_body(scores_hbm, labels_hbm, h2_hbm, ba_hbm,
                  cs_hbm, ci_hbm,
                  sc_v, lb_v, red_v, tmp_v, h2s_v, ba_v, t22_v,
                  cmp_s, cmp_i, cmp_l, loc_s, loc_i, cnt_v):
    cid = lax.axis_index("c")
    sid = lax.axis_index("s")
    wid = cid * 16 + sid
    lane = lax.iota(jnp.int32, 16)
    for part in range(16):
        pltpu.sync_copy(h2_hbm.at[0, pl.ds(part * _STRIPE, _STRIPE)], red_v)
        pltpu.sync_copy(h2_hbm.at[1, pl.ds(part * _STRIPE, _STRIPE)], tmp_v)

        def badd(i, c, _part=part):
            h2s_v[pl.ds(_part * _STRIPE + i * 16, 16)] = (
                red_v[pl.ds(i * 16, 16)] + tmp_v[pl.ds(i * 16, 16)])
            return c

        lax.fori_loop(0, _STRIPE // 16, badd, 0)
    pltpu.sync_copy(ba_hbm, ba_v)
    b_lo = ba_v[pl.ds(0, 16)]
    b_hi = ba_v[pl.ds(16, 16)]
    a_lo = ba_v[pl.ds(32, 16)]
    a_hi = ba_v[pl.ds(48, 16)]
    t_lo = jnp.zeros((16,), jnp.int32)
    t_hi = jnp.zeros((16,), jnp.int32)
    for k in range(_NCLASSES):
        if k < 16:
            ak = jnp.sum(jnp.where(lane == k, a_lo, 0))
            bk = jnp.sum(jnp.where(lane == k, b_lo, 0))
        else:
            ak = jnp.sum(jnp.where(lane == (k - 16), a_hi, 0))
            bk = jnp.sum(jnp.where(lane == (k - 16), b_hi, 0))
        b2k, _ = _scan_boundary(h2s_v, k, _SAMPLES - ak)
        t22k = bk * _NBKT + b2k
        if k < 16:
            t_lo = jnp.where(lane == k, t22k, t_lo)
        else:
            t_hi = jnp.where(lane == (k - 16), t22k, t_hi)
    t22_v[pl.ds(0, 16)] = t_lo
    t22_v[pl.ds(16, 16)] = t_hi
    neg = jnp.full((16,), -3e38, jnp.float32)
    zi = jnp.zeros((16,), jnp.int32)
    for k in range(_NCLASSES):
        loc_s[pl.ds(k * _CCAP, 16)] = neg
        loc_i[pl.ds(k * _CCAP, 16)] = zi
    cnt_v[pl.ds(0, 16)] = zi
    cnt_v[pl.ds(16, 16)] = zi
    L = jnp.int32(0)
    for ch in range(_SLICE // _CHUNK):
        gbase = wid * _SLICE + ch * _CHUNK
        _load_chunk(scores_hbm, labels_hbm, sc_v, lb_v, gbase)

        def b(i, L, _gbase=gbase):
            s = sc_v[pl.ds(i * 16, 16)]
            l = lb_v[pl.ds(i * 16, 16)]
            key20 = lax.shift_right_logical(_sortable_key(s), _SH2)
            t = plsc.load_gather(t22_v, [l])
            m = key20 >= t
            cnt = jnp.sum(m.astype(jnp.int32))
            Lc = jnp.minimum(L, _LCAP - 16)
            plsc.store_compressed(cmp_s.at[pl.ds(Lc, 16)], s, mask=m)
            plsc.store_compressed(cmp_i.at[pl.ds(Lc, 16)],
                                  _gbase + i * 16 + lane, mask=m)
            plsc.store_compressed(cmp_l.at[pl.ds(Lc, 16)], l, mask=m)
            return L + cnt

        L = lax.fori_loop(0, _CHUNK // 16, b, L)
    L = jnp.minimum(L, _LCAP - 16)

    def redis(e, c):
        ev = jnp.broadcast_to(e, (16,))
        k = jnp.max(plsc.load_gather(cmp_l, [ev]))
        sc = jnp.max(plsc.load_gather(cmp_s, [ev]))
        gi = jnp.max(plsc.load_gather(cmp_i, [ev]))
        ck = jnp.max(plsc.load_gather(cnt_v, [jnp.broadcast_to(k, (16,))]))
        ckc = jnp.minimum(ck, _CCAP - 1)
        m0 = lane == 0
        pos = jnp.broadcast_to(k * _CCAP + ckc, (16,))
        plsc.store_scatter(loc_s, [pos], jnp.broadcast_to(sc, (16,)), mask=m0)
        plsc.store_scatter(loc_i, [pos], jnp.broadcast_to(gi, (16,)), mask=m0)
        plsc.store_scatter(cnt_v, [jnp.broadcast_to(k, (16,))],
                           jnp.broadcast_to(ck + 1, (16,)), mask=m0)
        return c

    lax.fori_loop(0, L, redis, 0)
    for k in range(_NCLASSES):
        pltpu.sync_copy(loc_s.at[pl.ds(k * _CCAP, _CCAP)],
                        cs_hbm.at[k, pl.ds(wid * _CCAP, _CCAP)])
        pltpu.sync_copy(loc_i.at[pl.ds(k * _CCAP, _CCAP)],
                        ci_hbm.at[k, pl.ds(wid * _CCAP, _CCAP)])


def _sc_topk(scores_flat, labels_flat):
    mesh = plsc.VectorSubcoreMesh(core_axis_name="c", subcore_axis_name="s")
    cp = pltpu.CompilerParams(needs_layout_passes=False)
    data_scratch = [
        pltpu.VMEM((_CHUNK,), jnp.float32),
        pltpu.VMEM((_CHUNK,), jnp.int32),
    ]
    h1 = pl.kernel(
        _hist1_body, mesh=mesh, compiler_params=cp,
        out_type=jax.ShapeDtypeStruct((2, _HISTW), jnp.int32),
        scratch_types=data_scratch + [
            pltpu.VMEM((_HISTW,), jnp.int32),
            pltpu.VMEM((_STRIPE,), jnp.int32),
            pltpu.VMEM((_STRIPE,), jnp.int32),
            pltpu.VMEM_SHARED((16, _HISTW), jnp.int32),
        ],
    )(scores_flat, labels_flat)
    h2, ba = pl.kernel(
        _hist2_body, mesh=mesh, compiler_params=cp,
        out_type=[jax.ShapeDtypeStruct((2, _HISTW), jnp.int32),
                  jax.ShapeDtypeStruct((64,), jnp.int32)],
        scratch_types=data_scratch + [
            pltpu.VMEM((_HISTW,), jnp.int32),
            pltpu.VMEM((_STRIPE,), jnp.int32),
            pltpu.VMEM((_STRIPE,), jnp.int32),
            pltpu.VMEM((_HISTW,), jnp.int32),
            pltpu.VMEM((32,), jnp.int32),
            pltpu.VMEM((64,), jnp.int32),
            pltpu.VMEM_SHARED((16, _HISTW), jnp.int32),
        ],
    )(scores_flat, labels_flat, h1)
    cand_s, cand_i = pl.kernel(
        _collect_body, mesh=mesh, compiler_params=cp,
        out_type=[
            jax.ShapeDtypeStruct((_NCLASSES, _NW * _CCAP), jnp.float32),
            jax.ShapeDtypeStruct((_NCLASSES, _NW * _CCAP), jnp.int32),
        ],
        scratch_types=data_scratch + [
            pltpu.VMEM((_STRIPE,), jnp.int32),
            pltpu.VMEM((_STRIPE,), jnp.int32),
            pltpu.VMEM((_HISTW,), jnp.int32),
            pltpu.VMEM((64,), jnp.int32),
            pltpu.VMEM((32,), jnp.int32),
            pltpu.VMEM((_LCAP,), jnp.float32),
            pltpu.VMEM((_LCAP,), jnp.int32),
            pltpu.VMEM((_LCAP,), jnp.int32),
            pltpu.VMEM((_NCLASSES * _CCAP,), jnp.float32),
            pltpu.VMEM((_NCLASSES * _CCAP,), jnp.int32),
            pltpu.VMEM((32,), jnp.int32),
        ],
    )(scores_flat, labels_flat, h2, ba)
    return cand_s, cand_i


_CAND = _NW * _CCAP


def _rank_kernel(sr_ref, sc_ref, ir_ref, ic_ref, out_ref):
    sj = sr_ref[0]
    si = sc_ref[0]
    ij = ir_ref[0]
    ii = ic_ref[0]
    better = jnp.logical_or(sj > si, jnp.logical_and(sj == si, ij < ii))
    rank = jnp.sum(better.astype(jnp.float32), axis=1,
                   keepdims=True).astype(jnp.int32)
    sel = (rank == jax.lax.broadcasted_iota(
        jnp.int32, (_CAND, _SAMPLES), 1)).astype(jnp.float32)
    topi = jnp.sum(sel * ic_ref[0].astype(jnp.float32), axis=0)
    out_ref[0, 0, :] = topi.astype(jnp.int32)


def _rank_sort(cand_s, cand_i):
    row = lambda a: a.reshape(_NCLASSES, 1, _CAND)
    col = lambda a: a.reshape(_NCLASSES, _CAND, 1)
    out = pl.pallas_call(
        _rank_kernel,
        grid=(_NCLASSES,),
        in_specs=[
            pl.BlockSpec((1, 1, _CAND), lambda k: (k, 0, 0)),
            pl.BlockSpec((1, _CAND, 1), lambda k: (k, 0, 0)),
            pl.BlockSpec((1, 1, _CAND), lambda k: (k, 0, 0)),
            pl.BlockSpec((1, _CAND, 1), lambda k: (k, 0, 0)),
        ],
        out_specs=pl.BlockSpec((1, 1, _SAMPLES), lambda k: (k, 0, 0)),
        out_shape=jax.ShapeDtypeStruct((_NCLASSES, 1, _SAMPLES), jnp.int32),
    )(row(cand_s), col(cand_s), row(cand_i), col(cand_i))
    return out.reshape(_NCLASSES, _SAMPLES)


def _proto_kernel(lab_ref, ft_ref, sums_ref, counts_ref):
    b = pl.program_id(0)
    j = pl.program_id(1)

    @pl.when(jnp.logical_and(b == 0, j == 0))
    def _():
        sums_ref[...] = jnp.zeros_like(sums_ref)
        counts_ref[...] = jnp.zeros_like(counts_ref)

    lab = lab_ref[0, 0, :]
    oh = (lab[None, :] == jax.lax.broadcasted_iota(
        jnp.int32, (_NCLASSES, _BP), 0)).astype(jnp.float32)
    ft = ft_ref[0]
    sums_ref[:, :_NCLASSES] += jax.lax.dot_general(
        ft, oh, (((1,), (1,)), ((), ())))
    counts_ref[0, :_NCLASSES] += jnp.sum(oh, axis=1)


def _score_kernel(lab_ref, ft_ref, sums_ref, counts_ref, score_ref):
    proto = sums_ref[:, :_NCLASSES] / (counts_ref[0, :_NCLASSES] + 1e-6)
    ft = ft_ref[0]
    sim = jax.lax.dot_general(
        proto, ft, (((0,), (0,)), ((), ())))
    lab = lab_ref[0, 0, :]
    oh = (lab[None, :] == jax.lax.broadcasted_iota(
        jnp.int32, (_NCLASSES, _BP), 0)).astype(jnp.float32)
    score_ref[...] = jnp.sum(sim * oh, axis=0)


def _aff_kernel(s_ref, t_ref, out_ref):
    inv = 1.0 / math.sqrt(128.0)
    out_ref[0] = jax.lax.dot_general(
        s_ref[0], t_ref[0], (((1,), (1,)), ((), ()))) * inv


def kernel(feat_s, feat_t, label_t):
    bs, c, p = feat_s.shape
    nb = p // _BP
    lab3 = label_t.reshape(bs * nb, 1, _BP)

    sums, counts = pl.pallas_call(
        _proto_kernel,
        grid=(bs, nb),
        in_specs=[
            pl.BlockSpec((1, 1, _BP), lambda b, j: (b * nb + j, 0, 0)),
            pl.BlockSpec((1, c, _BP), lambda b, j: (b, 0, j)),
        ],
        out_specs=[
            pl.BlockSpec((c, 32), lambda b, j: (0, 0)),
            pl.BlockSpec((8, 32), lambda b, j: (0, 0)),
        ],
        out_shape=[
            jax.ShapeDtypeStruct((c, 32), jnp.float32),
            jax.ShapeDtypeStruct((8, 32), jnp.float32),
        ],
    )(lab3, feat_t)

    scores = pl.pallas_call(
        _score_kernel,
        grid=(bs, nb),
        in_specs=[
            pl.BlockSpec((1, 1, _BP), lambda b, j: (b * nb + j, 0, 0)),
            pl.BlockSpec((1, c, _BP), lambda b, j: (b, 0, j)),
            pl.BlockSpec((c, 32), lambda b, j: (0, 0)),
            pl.BlockSpec((8, 32), lambda b, j: (0, 0)),
        ],
        out_specs=pl.BlockSpec((_BP,), lambda b, j: (b * nb + j,)),
        out_shape=jax.ShapeDtypeStruct((bs * nb * _BP,), jnp.float32),
    )(lab3, feat_t, sums, counts)

    cand_s, cand_i = _sc_topk(scores, label_t.reshape(-1))
    topi = _rank_sort(cand_s, cand_i)

    def _physical_view(x):
        return x.reshape(bs, c // 8, 8, p // 128, 128).transpose(
            0, 1, 3, 2, 4).reshape(bs * c * p)

    samp_s, samp_t = _sc_gather(
        topi.reshape(-1), _physical_view(feat_s), _physical_view(feat_t))
    sampled_s = samp_s.reshape(_NCLASSES, _SAMPLES, c)
    sampled_t = samp_t.reshape(_NCLASSES, _SAMPLES, c)

    aff = pl.pallas_call(
        _aff_kernel,
        grid=(_NCLASSES,),
        in_specs=[
            pl.BlockSpec((1, _SAMPLES, c), lambda k: (k, 0, 0)),
            pl.BlockSpec((1, _SAMPLES, c), lambda k: (k, 0, 0)),
        ],
        out_specs=pl.BlockSpec((1, _SAMPLES, _SAMPLES), lambda k: (k, 0, 0)),
        out_shape=jax.ShapeDtypeStruct((_NCLASSES, _SAMPLES, _SAMPLES), jnp.float32),
    )(sampled_s, sampled_t)
    return aff


# speedup vs baseline: 1.7523x; 1.1054x over previous
import functools
import math

import jax
import jax.numpy as jnp
from jax import lax
from jax.experimental import pallas as pl
from jax.experimental.pallas import tpu as pltpu
from jax.experimental.pallas import tpu_sc as plsc

_NCLASSES = 20
_SAMPLES = 128
_BP = 2048
_NW = 32
_GPW = (_NCLASSES * _SAMPLES) // _NW
_PSTRIDE = 131072
_CH = 128


def _gather_body(topi_hbm, fs_hbm, ft_hbm, out_s, out_t,
                 idx_v, bases_v, chidx_v, rows_s, rows_t, sem_s, sem_t):
    wid = lax.axis_index("s") * 2 + lax.axis_index("c")
    base = wid * _GPW
    pltpu.sync_copy(topi_hbm.at[pl.ds(base, _GPW)], idx_v)
    lane = lax.iota(jnp.int32, 16)
    for t in range(_GPW // 16):
        nvec = idx_v[pl.ds(t * 16, 16)]
        bb = lax.shift_right_logical(nvec, 17)
        ii = jnp.bitwise_and(nvec, jnp.int32(_PSTRIDE - 1))
        bvec = (bb * (_CH * _PSTRIDE)
                + lax.shift_right_logical(ii, 7) * 1024
                + jnp.bitwise_and(ii, jnp.int32(127)))
        for b in range(16):
            j = t * 16 + b
            bj = jnp.broadcast_to(
                jnp.sum(jnp.where(lane == b, bvec, 0)), (16,))
            for g in range(8):
                cv = lane + g * 16
                chidx_v[j, pl.ds(g * 16, 16)] = (
                    bj + lax.shift_right_logical(cv, 3) * (_PSTRIDE * 8)
                    + jnp.bitwise_and(cv, jnp.int32(7)) * 128)
    for j in range(_GPW):
        pltpu.async_copy(fs_hbm.at[chidx_v.at[j]], rows_s.at[j], sem_s)
        pltpu.async_copy(ft_hbm.at[chidx_v.at[j]], rows_t.at[j], sem_t)
    for j in range(_GPW):
        pltpu.make_async_copy(fs_hbm.at[chidx_v.at[j]], rows_s.at[j],
                              sem_s).wait()
        pltpu.make_async_copy(ft_hbm.at[chidx_v.at[j]], rows_t.at[j],
                              sem_t).wait()
    pltpu.sync_copy(rows_s, out_s.at[pl.ds(base, _GPW)])
    pltpu.sync_copy(rows_t, out_t.at[pl.ds(base, _GPW)])


def _sc_gather(topi_flat, fs_flat, ft_flat):
    n_samp = _NCLASSES * _SAMPLES
    mesh = plsc.VectorSubcoreMesh(core_axis_name="c", subcore_axis_name="s")
    f = pl.kernel(
        _gather_body,
        mesh=mesh,
        compiler_params=pltpu.CompilerParams(needs_layout_passes=False),
        out_type=[
            jax.ShapeDtypeStruct((n_samp, _CH), jnp.float32),
            jax.ShapeDtypeStruct((n_samp, _CH), jnp.float32),
        ],
        scratch_types=[
            pltpu.VMEM((_GPW,), jnp.int32),
            pltpu.VMEM((_GPW,), jnp.int32),
            pltpu.VMEM((_GPW, _CH), jnp.int32),
            pltpu.VMEM((_GPW, _CH), jnp.float32),
            pltpu.VMEM((_GPW, _CH), jnp.float32),
            pltpu.SemaphoreType.DMA,
            pltpu.SemaphoreType.DMA,
        ],
    )
    return f(topi_flat, fs_flat, ft_flat)


_NBKT = 1024
_HISTW = _NCLASSES * _NBKT
_SLICE = 262144 // _NW
_CHUNK = 2048
_STRIPE = _HISTW // 16
_SH1 = 32 - 10
_SH2 = 32 - 20
_CCAP = 16
_LCAP = 240


def _sortable_key(s):
    k1 = plsc.bitcast(s, jnp.int32)
    return jnp.where(k1 < 0, ~k1, k1 | jnp.int32(-(2 ** 31)))


def _load_chunk(scores_hbm, labels_hbm, sc_v, lb_v, base):
    pltpu.sync_copy(scores_hbm.at[pl.ds(base, _CHUNK)], sc_v)
    pltpu.sync_copy(labels_hbm.at[pl.ds(base, _CHUNK)], lb_v)


def _zero_vmem_i32(ref, nwords):
    z = jnp.zeros((16,), jnp.int32)

    def b(i, c):
        ref[pl.ds(i * 16, 16)] = z
        return c

    lax.fori_loop(0, nwords // 16, b, 0)


def _publish_reduce(hist_v, spmem, red_v, stage_v, sem, sid, cid, out_hbm):
    pltpu.sync_copy(hist_v, spmem.at[sid])
    plsc.subcore_barrier()
    for r in range(16):
        pltpu.async_copy(spmem.at[r, pl.ds(sid * _STRIPE, _STRIPE)],
                         stage_v.at[r], sem)
    for r in range(16):
        pltpu.make_async_copy(spmem.at[r, pl.ds(sid * _STRIPE, _STRIPE)],
                              stage_v.at[r], sem).wait()

    def badd(i, c):
        acc = stage_v[0, pl.ds(i * 16, 16)]
        for r in range(1, 16):
            acc = acc + stage_v[r, pl.ds(i * 16, 16)]
        red_v[pl.ds(i * 16, 16)] = acc
        return c

    lax.fori_loop(0, _STRIPE // 16, badd, 0)
    pltpu.sync_copy(red_v, out_hbm.at[cid, pl.ds(sid * _STRIPE, _STRIPE)])


def _scan_boundary(hist_ref, k, target):
    lane = lax.iota(jnp.int32, 16)

    def cond(st):
        return jnp.logical_not(st[4])

    def body(st):
        v, cum, bb, aa, done = st
        hv = hist_ref[pl.ds(k * _NBKT + v * 16, 16)]
        rc = lax.rev(hv, (0,))
        cs = jnp.cumsum(rc)
        cum_incl = cum + cs
        mask = cum_incl >= target
        mask = jnp.logical_or(mask, jnp.logical_and(v == 0, lane == 15))
        anyhit = jnp.max(mask.astype(jnp.int32)) > 0
        f = jnp.max(plsc.all_reduce_ffs(mask))
        bnew = v * 16 + 15 - f
        csm1 = jnp.sum(jnp.where(lane == f - 1, cs, 0))
        anew = cum + csm1
        vec_total = jnp.sum(hv)
        return (jnp.where(anyhit, v, v - 1),
                jnp.where(anyhit, cum, cum + vec_total),
                jnp.where(anyhit, bnew, bb),
                jnp.where(anyhit, anew, aa),
                anyhit)

    st = lax.while_loop(cond, body, (jnp.int32(_NBKT // 16 - 1),
                                     jnp.int32(0), jnp.int32(0),
                                     jnp.int32(0), False))
    return st[2], st[3]


def _hist1_body(scores_hbm, labels_hbm, out_hbm,
                sc_v, lb_v, hist_v, red_v, stage_v, spmem, sem):
    cid = lax.axis_index("c")
    sid = lax.axis_index("s")
    wid = cid * 16 + sid
    _zero_vmem_i32(hist_v, _HISTW)
    ones = jnp.ones((16,), jnp.int32)
    for ch in range(_SLICE // _CHUNK):
        _load_chunk(scores_hbm, labels_hbm, sc_v, lb_v,
                    wid * _SLICE + ch * _CHUNK)

        def b(i, c):
            s = sc_v[pl.ds(i * 16, 16)]
            l = lb_v[pl.ds(i * 16, 16)]
            key = _sortable_key(s)
            b1 = lax.shift_right_logical(key, _SH1)
            plsc.addupdate_scatter(hist_v, [l * _NBKT + b1], ones)
            return c

        lax.fori_loop(0, _CHUNK // 16, b, 0)
    _publish_reduce(hist_v, spmem, red_v, stage_v, sem, sid, cid, out_hbm)


def _hist2_body(scores_hbm, labels_hbm, h1_hbm, out_hbm, ba_hbm,
                sc_v, lb_v, hist_v, red_v, stage_v, h1s_v, btab_v, ba_v,
                spmem, sem):
    cid = lax.axis_index("c")
    sid = lax.axis_index("s")
    wid = cid * 16 + sid
    lane = lax.iota(jnp.int32, 16)
    c0 = pltpu.async_copy(h1_hbm.at[0], h1s_v, sem)
    c1 = pltpu.async_copy(h1_hbm.at[1], hist_v, sem)
    c0.wait()
    c1.wait()

    def bsum(i, c):
        h1s_v[pl.ds(i * 16, 16)] = (h1s_v[pl.ds(i * 16, 16)]
                                    + hist_v[pl.ds(i * 16, 16)])
        return c

    lax.fori_loop(0, _HISTW // 16, bsum, 0)
    b_lo = jnp.zeros((16,), jnp.int32)
    b_hi = jnp.zeros((16,), jnp.int32)
    a_lo = jnp.zeros((16,), jnp.int32)
    a_hi = jnp.zeros((16,), jnp.int32)
    for k in range(_NCLASSES):
        bk, ak = _scan_boundary(h1s_v, k, _SAMPLES)
        if k < 16:
            b_lo = jnp.where(lane == k, bk, b_lo)
            a_lo = jnp.where(lane == k, ak, a_lo)
        else:
            b_hi = jnp.where(lane == (k - 16), bk, b_hi)
            a_hi = jnp.where(lane == (k - 16), ak, a_hi)
    btab_v[pl.ds(0, 16)] = b_lo
    btab_v[pl.ds(16, 16)] = b_hi
    ba_v[pl.ds(0, 16)] = b_lo
    ba_v[pl.ds(16, 16)] = b_hi
    ba_v[pl.ds(32, 16)] = a_lo
    ba_v[pl.ds(48, 16)] = a_hi

    @pl.when(wid == 0)
    def _():
        pltpu.sync_copy(ba_v, ba_hbm)

    _zero_vmem_i32(hist_v, _HISTW)
    ones = jnp.ones((16,), jnp.int32)
    for ch in range(_SLICE // _CHUNK):
        _load_chunk(scores_hbm, labels_hbm, sc_v, lb_v,
                    wid * _SLICE + ch * _CHUNK)

        def b(i, c):
            s = sc_v[pl.ds(i * 16, 16)]
            l = lb_v[pl.ds(i * 16, 16)]
            key = _sortable_key(s)
            b1 = lax.shift_right_logical(key, _SH1)
            sub = jnp.bitwise_and(lax.shift_right_logical(key, _SH2),
                                  jnp.int32(_NBKT - 1))
            bl = plsc.load_gather(btab_v, [l])
            m = b1 == bl
            plsc.addupdate_scatter(hist_v, [l * _NBKT + sub], ones, mask=m)
            return c

        lax.fori_loop(0, _CHUNK // 16, b, 0)
    _publish_reduce(hist_v, spmem, red_v, stage_v, sem, sid, cid, out_hbm)


def _collect_body(scores_hbm, labels_hbm, h2_hbm, ba_hbm,
                  cs_hbm, ci_hbm,
                  sc_v, lb_v, tmp2_v, h2s_v, ba_v, t22_v,
                  cmp_s, cmp_i, cmp_l, loc_s, loc_i, cnt_v, sem):
    cid = lax.axis_index("c")
    sid = lax.axis_index("s")
    wid = cid * 16 + sid
    lane = lax.iota(jnp.int32, 16)
    c0 = pltpu.async_copy(h2_hbm.at[0], h2s_v, sem)
    c1 = pltpu.async_copy(h2_hbm.at[1], tmp2_v, sem)
    c0.wait()
    c1.wait()

    def bsum(i, c):
        h2s_v[pl.ds(i * 16, 16)] = (h2s_v[pl.ds(i * 16, 16)]
                                    + tmp2_v[pl.ds(i * 16, 16)])
        return c

    lax.fori_loop(0, _HISTW // 16, bsum, 0)
    pltpu.sync_copy(ba_hbm, ba_v)
    b_lo = ba_v[pl.ds(0, 16)]
    b_hi = ba_v[pl.ds(16, 16)]
    a_lo = ba_v[pl.ds(32, 16)]
    a_hi = ba_v[pl.ds(48, 16)]
    t_lo = jnp.zeros((16,), jnp.int32)
    t_hi = jnp.zeros((16,), jnp.int32)
    for k in range(_NCLASSES):
        if k < 16:
            ak = jnp.sum(jnp.where(lane == k, a_lo, 0))
            bk = jnp.sum(jnp.where(lane == k, b_lo, 0))
        else:
            ak = jnp.sum(jnp.where(lane == (k - 16), a_hi, 0))
            bk = jnp.sum(jnp.where(lane == (k - 16), b_hi, 0))
        b2k, _ = _scan_boundary(h2s_v, k, _SAMPLES - ak)
        t22k = bk * _NBKT + b2k
        if k < 16:
            t_lo = jnp.where(lane == k, t22k, t_lo)
        else:
            t_hi = jnp.where(lane == (k - 16), t22k, t_hi)
    t22_v[pl.ds(0, 16)] = t_lo
    t22_v[pl.ds(16, 16)] = t_hi
    neg = jnp.full((16,), -3e38, jnp.float32)
    zi = jnp.zeros((16,), jnp.int32)
    for k in range(_NCLASSES):
        loc_s[pl.ds(k * _CCAP, 16)] = neg
        loc_i[pl.ds(k * _CCAP, 16)] = zi
    cnt_v[pl.ds(0, 16)] = zi
    cnt_v[pl.ds(16, 16)] = zi
    L = jnp.int32(0)
    for ch in range(_SLICE // _CHUNK):
        gbase = wid * _SLICE + ch * _CHUNK
        _load_chunk(scores_hbm, labels_hbm, sc_v, lb_v, gbase)

        def b(i, L, _gbase=gbase):
            s = sc_v[pl.ds(i * 16, 16)]
            l = lb_v[pl.ds(i * 16, 16)]
            key20 = lax.shift_right_logical(_sortable_key(s), _SH2)
            t = plsc.load_gather(t22_v, [l])
            m = key20 >= t
            cnt = jnp.sum(m.astype(jnp.int32))
            Lc = jnp.minimum(L, _LCAP - 16)
            plsc.store_compressed(cmp_s.at[pl.ds(Lc, 16)], s, mask=m)
            plsc.store_compressed(cmp_i.at[pl.ds(Lc, 16)],
                                  _gbase + i * 16 + lane, mask=m)
            plsc.store_compressed(cmp_l.at[pl.ds(Lc, 16)], l, mask=m)
            return L + cnt

        L = lax.fori_loop(0, _CHUNK // 16, b, L)
    L = jnp.minimum(L, _LCAP - 16)

    def redis(e, c):
        ev = jnp.broadcast_to(e, (16,))
        k = jnp.max(plsc.load_gather(cmp_l, [ev]))
        sc = jnp.max(plsc.load_gather(cmp_s, [ev]))
        gi = jnp.max(plsc.load_gather(cmp_i, [ev]))
        ck = jnp.max(plsc.load_gather(cnt_v, [jnp.broadcast_to(k, (16,))]))
        ckc = jnp.minimum(ck, _CCAP - 1)
        m0 = lane == 0
        pos = jnp.broadcast_to(k * _CCAP + ckc, (16,))
        plsc.store_scatter(loc_s, [pos], jnp.broadcast_to(sc, (16,)), mask=m0)
        plsc.store_scatter(loc_i, [pos], jnp.broadcast_to(gi, (16,)), mask=m0)
        plsc.store_scatter(cnt_v, [jnp.broadcast_to(k, (16,))],
                           jnp.broadcast_to(ck + 1, (16,)), mask=m0)
        return c

    lax.fori_loop(0, L, redis, 0)
    for k in range(_NCLASSES):
        pltpu.sync_copy(loc_s.at[pl.ds(k * _CCAP, _CCAP)],
                        cs_hbm.at[k, pl.ds(wid * _CCAP, _CCAP)])
        pltpu.sync_copy(loc_i.at[pl.ds(k * _CCAP, _CCAP)],
                        ci_hbm.at[k, pl.ds(wid * _CCAP, _CCAP)])


def _sc_topk(scores_flat, labels_flat):
    mesh = plsc.VectorSubcoreMesh(core_axis_name="c", subcore_axis_name="s")
    cp = pltpu.CompilerParams(needs_layout_passes=False)
    data_scratch = [
        pltpu.VMEM((_CHUNK,), jnp.float32),
        pltpu.VMEM((_CHUNK,), jnp.int32),
    ]
    h1 = pl.kernel(
        _hist1_body, mesh=mesh, compiler_params=cp,
        out_type=jax.ShapeDtypeStruct((2, _HISTW), jnp.int32),
        scratch_types=data_scratch + [
            pltpu.VMEM((_HISTW,), jnp.int32),
            pltpu.VMEM((_STRIPE,), jnp.int32),
            pltpu.VMEM((16, _STRIPE), jnp.int32),
            pltpu.VMEM_SHARED((16, _HISTW), jnp.int32),
            pltpu.SemaphoreType.DMA,
        ],
    )(scores_flat, labels_flat)
    h2, ba = pl.kernel(
        _hist2_body, mesh=mesh, compiler_params=cp,
        out_type=[jax.ShapeDtypeStruct((2, _HISTW), jnp.int32),
                  jax.ShapeDtypeStruct((64,), jnp.int32)],
        scratch_types=data_scratch + [
            pltpu.VMEM((_HISTW,), jnp.int32),
            pltpu.VMEM((_STRIPE,), jnp.int32),
            pltpu.VMEM((16, _STRIPE), jnp.int32),
            pltpu.VMEM((_HISTW,), jnp.int32),
            pltpu.VMEM((32,), jnp.int32),
            pltpu.VMEM((64,), jnp.int32),
            pltpu.VMEM_SHARED((16, _HISTW), jnp.int32),
            pltpu.SemaphoreType.DMA,
        ],
    )(scores_flat, labels_flat, h1)
    cand_s, cand_i = pl.kernel(
        _collect_body, mesh=mesh, compiler_params=cp,
        out_type=[
            jax.ShapeDtypeStruct((_NCLASSES, _NW * _CCAP), jnp.float32),
            jax.ShapeDtypeStruct((_NCLASSES, _NW * _CCAP), jnp.int32),
        ],
        scratch_types=data_scratch + [
            pltpu.VMEM((_HISTW,), jnp.int32),
            pltpu.VMEM((_HISTW,), jnp.int32),
            pltpu.VMEM((64,), jnp.int32),
            pltpu.VMEM((32,), jnp.int32),
            pltpu.VMEM((_LCAP,), jnp.float32),
            pltpu.VMEM((_LCAP,), jnp.int32),
            pltpu.VMEM((_LCAP,), jnp.int32),
            pltpu.VMEM((_NCLASSES * _CCAP,), jnp.float32),
            pltpu.VMEM((_NCLASSES * _CCAP,), jnp.int32),
            pltpu.VMEM((32,), jnp.int32),
            pltpu.SemaphoreType.DMA,
        ],
    )(scores_flat, labels_flat, h2, ba)
    return cand_s, cand_i


_CAND = _NW * _CCAP


def _rank_kernel(sr_ref, sc_ref, ir_ref, ic_ref, out_ref):
    sj = sr_ref[0]
    si = sc_ref[0]
    ij = ir_ref[0]
    ii = ic_ref[0]
    better = jnp.logical_or(sj > si, jnp.logical_and(sj == si, ij < ii))
    rank = jnp.sum(better.astype(jnp.float32), axis=1,
                   keepdims=True).astype(jnp.int32)
    sel = (rank == jax.lax.broadcasted_iota(
        jnp.int32, (_CAND, _SAMPLES), 1)).astype(jnp.float32)
    topi = jnp.sum(sel * ic_ref[0].astype(jnp.float32), axis=0)
    out_ref[0, 0, :] = topi.astype(jnp.int32)


def _rank_sort(cand_s, cand_i):
    row = lambda a: a.reshape(_NCLASSES, 1, _CAND)
    col = lambda a: a.reshape(_NCLASSES, _CAND, 1)
    out = pl.pallas_call(
        _rank_kernel,
        grid=(_NCLASSES,),
        in_specs=[
            pl.BlockSpec((1, 1, _CAND), lambda k: (k, 0, 0)),
            pl.BlockSpec((1, _CAND, 1), lambda k: (k, 0, 0)),
            pl.BlockSpec((1, 1, _CAND), lambda k: (k, 0, 0)),
            pl.BlockSpec((1, _CAND, 1), lambda k: (k, 0, 0)),
        ],
        out_specs=pl.BlockSpec((1, 1, _SAMPLES), lambda k: (k, 0, 0)),
        out_shape=jax.ShapeDtypeStruct((_NCLASSES, 1, _SAMPLES), jnp.int32),
    )(row(cand_s), col(cand_s), row(cand_i), col(cand_i))
    return out.reshape(_NCLASSES, _SAMPLES)


def _proto_kernel(lab_ref, ft_ref, sums_ref, counts_ref):
    b = pl.program_id(0)
    j = pl.program_id(1)

    @pl.when(jnp.logical_and(b == 0, j == 0))
    def _():
        sums_ref[...] = jnp.zeros_like(sums_ref)
        counts_ref[...] = jnp.zeros_like(counts_ref)

    lab = lab_ref[0, 0, :]
    oh = (lab[None, :] == jax.lax.broadcasted_iota(
        jnp.int32, (_NCLASSES, _BP), 0)).astype(jnp.float32)
    ft = ft_ref[0]
    sums_ref[:, :_NCLASSES] += jax.lax.dot_general(
        ft, oh, (((1,), (1,)), ((), ())))
    counts_ref[0, :_NCLASSES] += jnp.sum(oh, axis=1)


def _score_kernel(lab_ref, ft_ref, sums_ref, counts_ref, score_ref):
    proto = sums_ref[:, :_NCLASSES] / (counts_ref[0, :_NCLASSES] + 1e-6)
    ft = ft_ref[0]
    sim = jax.lax.dot_general(
        proto, ft, (((0,), (0,)), ((), ())))
    lab = lab_ref[0, 0, :]
    oh = (lab[None, :] == jax.lax.broadcasted_iota(
        jnp.int32, (_NCLASSES, _BP), 0)).astype(jnp.float32)
    score_ref[...] = jnp.sum(sim * oh, axis=0)


def _aff_kernel(s_ref, t_ref, out_ref):
    inv = 1.0 / math.sqrt(128.0)
    out_ref[0] = jax.lax.dot_general(
        s_ref[0], t_ref[0], (((1,), (1,)), ((), ()))) * inv


def kernel(feat_s, feat_t, label_t):
    bs, c, p = feat_s.shape
    nb = p // _BP
    lab3 = label_t.reshape(bs * nb, 1, _BP)

    sums, counts = pl.pallas_call(
        _proto_kernel,
        grid=(bs, nb),
        in_specs=[
            pl.BlockSpec((1, 1, _BP), lambda b, j: (b * nb + j, 0, 0)),
            pl.BlockSpec((1, c, _BP), lambda b, j: (b, 0, j)),
        ],
        out_specs=[
            pl.BlockSpec((c, 32), lambda b, j: (0, 0)),
            pl.BlockSpec((8, 32), lambda b, j: (0, 0)),
        ],
        out_shape=[
            jax.ShapeDtypeStruct((c, 32), jnp.float32),
            jax.ShapeDtypeStruct((8, 32), jnp.float32),
        ],
    )(lab3, feat_t)

    scores = pl.pallas_call(
        _score_kernel,
        grid=(bs, nb),
        in_specs=[
            pl.BlockSpec((1, 1, _BP), lambda b, j: (b * nb + j, 0, 0)),
            pl.BlockSpec((1, c, _BP), lambda b, j: (b, 0, j)),
            pl.BlockSpec((c, 32), lambda b, j: (0, 0)),
            pl.BlockSpec((8, 32), lambda b, j: (0, 0)),
        ],
        out_specs=pl.BlockSpec((_BP,), lambda b, j: (b * nb + j,)),
        out_shape=jax.ShapeDtypeStruct((bs * nb * _BP,), jnp.float32),
    )(lab3, feat_t, sums, counts)

    cand_s, cand_i = _sc_topk(scores, label_t.reshape(-1))
    topi = _rank_sort(cand_s, cand_i)

    def _physical_view(x):
        return x.reshape(bs, c // 8, 8, p // 128, 128).transpose(
            0, 1, 3, 2, 4).reshape(bs * c * p)

    samp_s, samp_t = _sc_gather(
        topi.reshape(-1), _physical_view(feat_s), _physical_view(feat_t))
    sampled_s = samp_s.reshape(_NCLASSES, _SAMPLES, c)
    sampled_t = samp_t.reshape(_NCLASSES, _SAMPLES, c)

    aff = pl.pallas_call(
        _aff_kernel,
        grid=(_NCLASSES,),
        in_specs=[
            pl.BlockSpec((1, _SAMPLES, c), lambda k: (k, 0, 0)),
            pl.BlockSpec((1, _SAMPLES, c), lambda k: (k, 0, 0)),
        ],
        out_specs=pl.BlockSpec((1, _SAMPLES, _SAMPLES), lambda k: (k, 0, 0)),
        out_shape=jax.ShapeDtypeStruct((_NCLASSES, _SAMPLES, _SAMPLES), jnp.float32),
    )(sampled_s, sampled_t)
    return aff
